# baseline (speedup 1.0000x reference)
import functools

import jax
import jax.numpy as jnp
from jax import lax
from jax.experimental import pallas as pl
from jax.experimental.pallas import tpu as pltpu
from jax.experimental.pallas import tpu_sc as plsc

_B, _C, _K, _D = 4096, 20, 20, 64
_V2 = 500000
_NC, _NS = 2, 16
_NW = _NC * _NS
_EPW = _B // _NW
_E = 16
_NCHUNK = _EPW // _E
_P = _E * _C
_GSUB = 4
_SUB = _P // _GSUB


_NWIN = 7812
_NBUF = 4
_NCHK = 62


def _sc_reformat(ct_t, ng_t, tail_c, tail_n):
    mesh = plsc.VectorSubcoreMesh(core_axis_name="c", subcore_axis_name="s")

    @functools.partial(
        pl.kernel,
        out_type=(jax.ShapeDtypeStruct((_V2, 2 * _D), jnp.float32),
                  jax.ShapeDtypeStruct((_V2, 2 * _D), jnp.float32)),
        mesh=mesh,
        scratch_types=(
            [pltpu.VMEM((_D, 128), jnp.float32) for _ in range(_NBUF)]
            + [pltpu.VMEM((64, 128), jnp.float32) for _ in range(_NBUF)]
            + [pltpu.SemaphoreType.DMA for _ in range(2 * _NBUF)]
        ),
        compiler_params=pltpu.CompilerParams(
            use_tc_tiling_on_sc=True, needs_layout_passes=False),
    )
    def k(ct_hbm, ng_hbm, tc_hbm, tn_hbm, co_hbm, no_hbm, *bufs):
        slabs = bufs[:_NBUF]
        rows = bufs[_NBUF:2 * _NBUF]
        sin = bufs[2 * _NBUF:2 * _NBUF + _NBUF]
        sout = bufs[2 * _NBUF + _NBUF:]
        wid = lax.axis_index("s") * _NC + lax.axis_index("c")
        lane16 = lax.iota(jnp.int32, 16)

        @pl.when(wid == 0)
        def _():
            pltpu.sync_copy(tc_hbm, rows[0].at[pl.ds(0, 32)])
            pltpu.sync_copy(rows[0].at[pl.ds(0, 32)],
                            co_hbm.at[pl.ds(_V2 - 32, 32)])
            pltpu.sync_copy(tn_hbm, rows[0].at[pl.ds(0, 32)])
            pltpu.sync_copy(rows[0].at[pl.ds(0, 32)],
                            no_hbm.at[pl.ds(_V2 - 32, 32)])

        def one_table(src_hbm, dst_hbm):
            def chunk(T, carry):
                for b in range(_NBUF):
                    widx = wid + 32 * (T * _NBUF + b)
                    w0 = pl.multiple_of(widx * 128, 128)

                    @pl.when(widx < _NWIN)
                    def _():
                        pltpu.async_copy(
                            src_hbm.at[:, pl.ds(w0, 128)], slabs[b], sin[b])
                for b in range(_NBUF):
                    widx = wid + 32 * (T * _NBUF + b)
                    w0 = pl.multiple_of(widx * 128, 128)

                    r0 = pl.multiple_of(widx * 64, 64)

                    @pl.when(widx < _NWIN)
                    def _():
                        pltpu.make_async_copy(
                            src_hbm.at[:, pl.ds(w0, 128)], slabs[b],
                            sin[b]).wait()

                        @pl.when(T > 0)
                        def _():
                            pltpu.make_async_copy(
                                rows[b], dst_hbm.at[pl.ds(r0, 64)],
                                sout[b]).wait()

                        def prow(p, pcarry):
                            for h in range(2):
                                wsp = jnp.full((16,), 0, jnp.int32) + (
                                    2 * p + h)
                                for d4 in range(_D // 16):
                                    v = plsc.load_gather(
                                        slabs[b], [d4 * 16 + lane16, wsp])
                                    rows[b][p, pl.ds(h * 64 + d4 * 16, 16)] = v
                            return pcarry
                        lax.fori_loop(0, 64, prow, 0)
                        pltpu.async_copy(
                            rows[b], dst_hbm.at[pl.ds(r0, 64)], sout[b])
                return carry
            lax.fori_loop(0, _NCHK, chunk, 0)
            for b in range(_NBUF):
                r0 = pl.multiple_of((wid + 32 * b) * 64, 64)
                pltpu.make_async_copy(
                    rows[b], dst_hbm.at[pl.ds(r0, 64)], sout[b]).wait()

        one_table(ct_hbm, co_hbm)
        one_table(ng_hbm, no_hbm)

    return k(ct_t, ng_t, tail_c, tail_n)


def _sc_pred(ctx_idx, foc_idx, cemb2, nemb2):
    mesh = plsc.VectorSubcoreMesh(core_axis_name="c", subcore_axis_name="s")

    @functools.partial(
        pl.kernel,
        out_type=jax.ShapeDtypeStruct((_B * _K,), jnp.float32),
        mesh=mesh,
        scratch_types=[
            pltpu.VMEM((_P,), jnp.int32),
            pltpu.VMEM((_P,), jnp.int32),
            pltpu.VMEM((_P,), jnp.int32),
            pltpu.VMEM((_P,), jnp.int32),
            pltpu.VMEM((_P,), jnp.int32),
            pltpu.VMEM((_P,), jnp.int32),
            pltpu.VMEM((_P, 2 * _D), jnp.float32),
            pltpu.VMEM((_P, 2 * _D), jnp.float32),
            pltpu.VMEM((_E, _D), jnp.float32),
            pltpu.VMEM((_P,), jnp.float32),
            pltpu.SemaphoreType.DMA,
            pltpu.SemaphoreType.DMA,
        ],
        compiler_params=pltpu.CompilerParams(
            use_tc_tiling_on_sc=False, needs_layout_passes=False),
    )
    def k(ci_hbm, fi_hbm, ce_hbm, ne_hbm, pred_hbm,
          ci_v, fi_v, cu_v, fu_v, ch_v, fh_v, cr_v, tr_v, src_v, pr_v,
          sem1, sem2):
        wid = lax.axis_index("s") * _NC + lax.axis_index("c")
        lane = lax.iota(jnp.int32, 16)

        def chunk_body(c, carry):
            po = (wid * _EPW + c * _E) * _C
            pltpu.sync_copy(ci_hbm.at[pl.ds(po, _P)], ci_v)
            pltpu.sync_copy(fi_hbm.at[pl.ds(po, _P)], fi_v)

            def ibody(i, icarry):
                sl = pl.ds(i * 16, 16)
                cw = ci_v[sl]
                cu_v[sl] = cw >> 1
                ch_v[sl] = (cw & 1) << 6
                fw = fi_v[sl]
                fu_v[sl] = fw >> 1
                fh_v[sl] = (fw & 1) << 6
                return icarry
            lax.fori_loop(0, _P // 16, ibody, 0)

            copies = []
            for i in range(_GSUB):
                s = pl.ds(i * _SUB, _SUB)
                copies.append(
                    pltpu.async_copy(ce_hbm.at[cu_v.at[s]], cr_v.at[s], sem1))
                copies.append(
                    pltpu.async_copy(ne_hbm.at[fu_v.at[s]], tr_v.at[s], sem2))
            for cp in copies:
                cp.wait()

            def ebody(e, ecarry):
                base = e * _C
                ha = ch_v[pl.ds(base, 16)]
                hb = ch_v[pl.ds(base + 4, 16)]
                for d4 in range(_D // 16):
                    h0 = ha[0]
                    acc = cr_v[base, pl.ds(h0 + d4 * 16, 16)]
                    for cc in range(1, _C):
                        h = ha[cc] if cc < 16 else hb[cc - 4]
                        acc = acc + cr_v[base + cc, pl.ds(h + d4 * 16, 16)]
                    src_v[e, pl.ds(d4 * 16, 16)] = acc
                return ecarry
            lax.fori_loop(0, _E, ebody, 0)

            def gbody(g, gcarry):
                row = g * 16 + lane
                b_loc = row // _K
                th = plsc.load_gather(fh_v, [row])
                acc = jnp.zeros((16,), jnp.float32)
                for d in range(_D):
                    dsp = jnp.full((16,), d, jnp.int32)
                    s = plsc.load_gather(src_v, [b_loc, dsp])
                    t = plsc.load_gather(tr_v, [row, th + dsp])
                    acc = acc + s * t
                pr_v[pl.ds(g * 16, 16)] = acc
                return gcarry
            lax.fori_loop(0, _P // 16, gbody, 0)

            pltpu.sync_copy(pr_v, pred_hbm.at[pl.ds(po, _P)])
            return carry

        lax.fori_loop(0, _NCHUNK, chunk_body, 0)

    return k(ctx_idx, foc_idx, cemb2, nemb2)


def _tc_loss_body(p_ref, w_ref, l_ref, o_ref):
    p = p_ref[...]
    w = w_ref[...]
    lbl = l_ref[...]
    bce = jnp.maximum(p, 0.0) - p * lbl + jnp.log1p(jnp.exp(-jnp.abs(p)))
    num = jnp.sum(w * bce, axis=1, keepdims=True)
    den = jnp.sum(w, axis=1, keepdims=True)
    o_ref[...] = jnp.sum(num / den, axis=0, keepdims=True) / p_ref.shape[0]


def kernel(input, focus_word, weight_mask, labels, context_emb, neg_emb):
    ci = input.reshape(-1)
    fi = focus_word.reshape(-1)
    ct2, nt2 = _sc_reformat(
        context_emb.T, neg_emb.T,
        context_emb[_NWIN * 128:].reshape(32, 2 * _D),
        neg_emb[_NWIN * 128:].reshape(32, 2 * _D))
    pred = _sc_pred(ci, fi, ct2, nt2)
    loss = pl.pallas_call(
        _tc_loss_body,
        out_shape=jax.ShapeDtypeStruct((1, 1), jnp.float32),
    )(pred.reshape(_B, _K), weight_mask, labels)
    return loss[0, 0]


# speedup vs baseline: 2.6479x; 2.6479x over previous
import functools

import jax
import jax.numpy as jnp
from jax import lax
from jax.experimental import pallas as pl
from jax.experimental.pallas import tpu as pltpu
from jax.experimental.pallas import tpu_sc as plsc

_B, _C, _K, _D = 4096, 20, 20, 64
_V2 = 500000
_NC, _NS = 2, 16
_NW = _NC * _NS
_EPW = _B // _NW
_E = 16
_NCHUNK = _EPW // _E
_P = _E * _C
_GSUB = 4
_SUB = _P // _GSUB


def _sc_pred(ctx_idx, foc_idx, cemb2, nemb2):
    mesh = plsc.VectorSubcoreMesh(core_axis_name="c", subcore_axis_name="s")

    @functools.partial(
        pl.kernel,
        out_type=jax.ShapeDtypeStruct((_B * _K,), jnp.float32),
        mesh=mesh,
        scratch_types=[
            pltpu.VMEM((_P,), jnp.int32),
            pltpu.VMEM((_P,), jnp.int32),
            pltpu.VMEM((_P,), jnp.int32),
            pltpu.VMEM((_P,), jnp.int32),
            pltpu.VMEM((_P,), jnp.int32),
            pltpu.VMEM((_P,), jnp.int32),
            pltpu.VMEM((_P, 2 * _D), jnp.float32),
            pltpu.VMEM((_P, 2 * _D), jnp.float32),
            pltpu.VMEM((_E, _D), jnp.float32),
            pltpu.VMEM((_P,), jnp.float32),
            pltpu.SemaphoreType.DMA,
            pltpu.SemaphoreType.DMA,
        ],
        compiler_params=pltpu.CompilerParams(
            use_tc_tiling_on_sc=False, needs_layout_passes=False),
    )
    def k(ci_hbm, fi_hbm, ce_hbm, ne_hbm, pred_hbm,
          ci_v, fi_v, cu_v, fu_v, ch_v, fh_v, cr_v, tr_v, src_v, pr_v,
          sem1, sem2):
        wid = lax.axis_index("s") * _NC + lax.axis_index("c")
        lane = lax.iota(jnp.int32, 16)

        def chunk_body(c, carry):
            po = (wid * _EPW + c * _E) * _C
            pltpu.sync_copy(ci_hbm.at[pl.ds(po, _P)], ci_v)
            pltpu.sync_copy(fi_hbm.at[pl.ds(po, _P)], fi_v)

            def ibody(i, icarry):
                sl = pl.ds(i * 16, 16)
                cw = ci_v[sl]
                cu_v[sl] = cw >> 1
                ch_v[sl] = (cw & 1) << 6
                fw = fi_v[sl]
                fu_v[sl] = fw >> 1
                fh_v[sl] = (fw & 1) << 6
                return icarry
            lax.fori_loop(0, _P // 16, ibody, 0)

            copies = []
            for i in range(_GSUB):
                s = pl.ds(i * _SUB, _SUB)
                copies.append(
                    pltpu.async_copy(ce_hbm.at[cu_v.at[s]], cr_v.at[s], sem1))
                copies.append(
                    pltpu.async_copy(ne_hbm.at[fu_v.at[s]], tr_v.at[s], sem2))
            for cp in copies:
                cp.wait()

            def ebody(e, ecarry):
                base = e * _C
                ha = ch_v[pl.ds(base, 16)]
                hb = ch_v[pl.ds(base + 4, 16)]
                for d4 in range(_D // 16):
                    h0 = ha[0]
                    acc = cr_v[base, pl.ds(h0 + d4 * 16, 16)]
                    for cc in range(1, _C):
                        h = ha[cc] if cc < 16 else hb[cc - 4]
                        acc = acc + cr_v[base + cc, pl.ds(h + d4 * 16, 16)]
                    src_v[e, pl.ds(d4 * 16, 16)] = acc
                return ecarry
            lax.fori_loop(0, _E, ebody, 0)

            def gbody(g, gcarry):
                row = g * 16 + lane
                b_loc = row // _K
                th = plsc.load_gather(fh_v, [row])
                acc = jnp.zeros((16,), jnp.float32)
                for d in range(_D):
                    dsp = jnp.full((16,), d, jnp.int32)
                    s = plsc.load_gather(src_v, [b_loc, dsp])
                    t = plsc.load_gather(tr_v, [row, th + dsp])
                    acc = acc + s * t
                pr_v[pl.ds(g * 16, 16)] = acc
                return gcarry
            lax.fori_loop(0, _P // 16, gbody, 0)

            pltpu.sync_copy(pr_v, pred_hbm.at[pl.ds(po, _P)])
            return carry

        lax.fori_loop(0, _NCHUNK, chunk_body, 0)

    return k(ctx_idx, foc_idx, cemb2, nemb2)


def _tc_loss_body(p_ref, w_ref, l_ref, o_ref):
    p = p_ref[...]
    w = w_ref[...]
    lbl = l_ref[...]
    bce = jnp.maximum(p, 0.0) - p * lbl + jnp.log1p(jnp.exp(-jnp.abs(p)))
    num = jnp.sum(w * bce, axis=1, keepdims=True)
    den = jnp.sum(w, axis=1, keepdims=True)
    o_ref[...] = jnp.sum(num / den, axis=0, keepdims=True) / p_ref.shape[0]


def kernel(input, focus_word, weight_mask, labels, context_emb, neg_emb):
    ci = input.reshape(-1)
    fi = focus_word.reshape(-1)
    ct2 = context_emb.reshape(_V2, 2 * _D)
    nt2 = neg_emb.reshape(_V2, 2 * _D)
    pred = _sc_pred(ci, fi, ct2, nt2)
    loss = pl.pallas_call(
        _tc_loss_body,
        out_shape=jax.ShapeDtypeStruct((1, 1), jnp.float32),
    )(pred.reshape(_B, _K), weight_mask, labels)
    return loss[0, 0]
